# baseline (speedup 1.0000x reference)
import jax
import jax.numpy as jnp
from jax.experimental import pallas as pl
from jax.experimental.pallas import tpu as pltpu

D_MODEL = 768
N_EXPERTS = 64
TOP_K = 8
BLOCK = 2048


def _tree(a, op):
    n = a.shape[0]
    while n > 1:
        h = n // 2
        a = op(a[:h], a[h:])
        n = h
    return a


def _router_body(x_ref, w_ref, bias_ref, wT_out_ref, idxT_out_ref,
                 counts_ref, stats_ref):
    i = pl.program_id(0)
    nsteps = pl.num_programs(0)

    w = w_ref[...]
    x = x_ref[...]
    lt = jax.lax.dot_general(w, x, (((1,), (1,)), ((), ())),
                             preferred_element_type=jnp.float32)
    lt = lt + bias_ref[...]

    sub_f = jax.lax.broadcasted_iota(jnp.int32, (N_EXPERTS, BLOCK), 0).astype(
        jnp.float32)
    cur = lt
    m_rows = []
    idx_rows = []
    for k in range(TOP_K):
        m = _tree(cur, jnp.maximum)
        t = cur == m
        idx = _tree(jnp.where(t, sub_f, jnp.float32(N_EXPERTS)),
                    jnp.minimum)
        m_rows.append(m)
        idx_rows.append(idx)
        cur = jnp.where(t & (sub_f == idx), -jnp.inf, cur)

    vals = jnp.concatenate(m_rows, axis=0)
    e = jnp.exp(vals - vals[:1])
    wT_out_ref[...] = e / _tree(e, jnp.add)
    idxT_out_ref[...] = jnp.concatenate(idx_rows, axis=0).astype(jnp.int32)

    block_counts = jnp.sum(jnp.where(cur == -jnp.inf, 1.0, 0.0),
                           axis=1, keepdims=True)

    @pl.when(i == 0)
    def _init():
        counts_ref[...] = block_counts

    @pl.when(i != 0)
    def _acc():
        counts_ref[...] = counts_ref[...] + block_counts

    @pl.when(i == nsteps - 1)
    def _stats():
        c = counts_ref[...]
        mean = _tree(c, jnp.add) / N_EXPERTS
        d = c - mean
        var = _tree(d * d, jnp.add) / (N_EXPERTS - 1)
        lb = jnp.sqrt(var) / (mean + 1e-6)
        cmax = _tree(c, jnp.maximum)
        cmin = _tree(c, jnp.minimum)
        l0 = jax.lax.broadcasted_iota(jnp.int32, (1, 128), 1)
        stats_ref[...] = (jnp.where(l0 == 0, lb, 0.0)
                          + jnp.where(l0 == 1, cmax, 0.0)
                          + jnp.where(l0 == 2, cmin, 0.0))


def kernel(x, W, expert_bias):
    b, s, d = x.shape
    nt = b * s
    x_flat = x.reshape(nt, d)
    bias = expert_bias.reshape(N_EXPERTS, 1)

    grid = (nt // BLOCK,)
    wT_out, idxT_out, counts, stats = pl.pallas_call(
        _router_body,
        grid=grid,
        in_specs=[
            pl.BlockSpec((BLOCK, d), lambda i: (i, 0)),
            pl.BlockSpec((N_EXPERTS, d), lambda i: (0, 0)),
            pl.BlockSpec((N_EXPERTS, 1), lambda i: (0, 0)),
        ],
        out_specs=[
            pl.BlockSpec((TOP_K, BLOCK), lambda i: (0, i)),
            pl.BlockSpec((TOP_K, BLOCK), lambda i: (0, i)),
            pl.BlockSpec((N_EXPERTS, 1), lambda i: (0, 0)),
            pl.BlockSpec((1, 128), lambda i: (0, 0)),
        ],
        out_shape=[
            jax.ShapeDtypeStruct((TOP_K, nt), jnp.float32),
            jax.ShapeDtypeStruct((TOP_K, nt), jnp.int32),
            jax.ShapeDtypeStruct((N_EXPERTS, 1), jnp.float32),
            jax.ShapeDtypeStruct((1, 128), jnp.float32),
        ],
        compiler_params=pltpu.CompilerParams(
            dimension_semantics=("arbitrary",),
        ),
    )(x_flat, W, bias)

    routing_weights = wT_out.T.reshape(b, s, TOP_K)
    expert_indices = idxT_out.T.reshape(b, s, TOP_K)
    expert_counts = counts[:, 0]
    load_balance = stats[0, 0]
    cmax = stats[0, 1]
    cmin = stats[0, 2]
    expected_load = jnp.asarray(nt * TOP_K / N_EXPERTS, dtype=jnp.float32)
    return (routing_weights, expert_indices, expert_counts, load_balance,
            cmax, cmin, expected_load)


# speedup vs baseline: 1.0256x; 1.0256x over previous
import jax
import jax.numpy as jnp
from jax.experimental import pallas as pl
from jax.experimental.pallas import tpu as pltpu

D_MODEL = 768
N_EXPERTS = 64
TOP_K = 8
BLOCK = 8192


def _tree(a, op):
    n = a.shape[0]
    while n > 1:
        h = n // 2
        a = op(a[:h], a[h:])
        n = h
    return a


def _router_body(x_ref, w_ref, bias_ref, wT_out_ref, idxT_out_ref,
                 counts_ref, stats_ref):
    i = pl.program_id(0)
    nsteps = pl.num_programs(0)

    w = w_ref[...]
    x = x_ref[...]
    lt = jax.lax.dot_general(w, x, (((1,), (1,)), ((), ())),
                             preferred_element_type=jnp.float32)
    lt = lt + bias_ref[...]

    sub_f = jax.lax.broadcasted_iota(jnp.int32, (N_EXPERTS, BLOCK), 0).astype(
        jnp.float32)
    cur = lt
    m_rows = []
    idx_rows = []
    for k in range(TOP_K):
        m = _tree(cur, jnp.maximum)
        t = cur == m
        idx = _tree(jnp.where(t, sub_f, jnp.float32(N_EXPERTS)),
                    jnp.minimum)
        m_rows.append(m)
        idx_rows.append(idx)
        cur = jnp.where(t & (sub_f == idx), -jnp.inf, cur)

    vals = jnp.concatenate(m_rows, axis=0)
    e = jnp.exp(vals - vals[:1])
    wT_out_ref[...] = e / _tree(e, jnp.add)
    idxT_out_ref[...] = jnp.concatenate(idx_rows, axis=0).astype(jnp.int32)

    block_counts = jnp.sum(jnp.where(cur == -jnp.inf, 1.0, 0.0),
                           axis=1, keepdims=True)

    @pl.when(i == 0)
    def _init():
        counts_ref[...] = block_counts

    @pl.when(i != 0)
    def _acc():
        counts_ref[...] = counts_ref[...] + block_counts

    @pl.when(i == nsteps - 1)
    def _stats():
        c = counts_ref[...]
        mean = _tree(c, jnp.add) / N_EXPERTS
        d = c - mean
        var = _tree(d * d, jnp.add) / (N_EXPERTS - 1)
        lb = jnp.sqrt(var) / (mean + 1e-6)
        cmax = _tree(c, jnp.maximum)
        cmin = _tree(c, jnp.minimum)
        l0 = jax.lax.broadcasted_iota(jnp.int32, (1, 128), 1)
        stats_ref[...] = (jnp.where(l0 == 0, lb, 0.0)
                          + jnp.where(l0 == 1, cmax, 0.0)
                          + jnp.where(l0 == 2, cmin, 0.0))


def kernel(x, W, expert_bias):
    b, s, d = x.shape
    nt = b * s
    x_flat = x.reshape(nt, d)
    bias = expert_bias.reshape(N_EXPERTS, 1)

    grid = (nt // BLOCK,)
    wT_out, idxT_out, counts, stats = pl.pallas_call(
        _router_body,
        grid=grid,
        in_specs=[
            pl.BlockSpec((BLOCK, d), lambda i: (i, 0)),
            pl.BlockSpec((N_EXPERTS, d), lambda i: (0, 0)),
            pl.BlockSpec((N_EXPERTS, 1), lambda i: (0, 0)),
        ],
        out_specs=[
            pl.BlockSpec((TOP_K, BLOCK), lambda i: (0, i)),
            pl.BlockSpec((TOP_K, BLOCK), lambda i: (0, i)),
            pl.BlockSpec((N_EXPERTS, 1), lambda i: (0, 0)),
            pl.BlockSpec((1, 128), lambda i: (0, 0)),
        ],
        out_shape=[
            jax.ShapeDtypeStruct((TOP_K, nt), jnp.float32),
            jax.ShapeDtypeStruct((TOP_K, nt), jnp.int32),
            jax.ShapeDtypeStruct((N_EXPERTS, 1), jnp.float32),
            jax.ShapeDtypeStruct((1, 128), jnp.float32),
        ],
        compiler_params=pltpu.CompilerParams(
            dimension_semantics=("arbitrary",),
        ),
    )(x_flat, W, bias)

    routing_weights = wT_out.T.reshape(b, s, TOP_K)
    expert_indices = idxT_out.T.reshape(b, s, TOP_K)
    expert_counts = counts[:, 0]
    load_balance = stats[0, 0]
    cmax = stats[0, 1]
    cmin = stats[0, 2]
    expected_load = jnp.asarray(nt * TOP_K / N_EXPERTS, dtype=jnp.float32)
    return (routing_weights, expert_indices, expert_counts, load_balance,
            cmax, cmin, expected_load)


# speedup vs baseline: 1.0789x; 1.0519x over previous
import jax
import jax.numpy as jnp
from jax.experimental import pallas as pl
from jax.experimental.pallas import tpu as pltpu

D_MODEL = 768
N_EXPERTS = 64
TOP_K = 8
BLOCK = 4096


def _tree(a, op):
    n = a.shape[0]
    while n > 1:
        h = n // 2
        a = op(a[:h], a[h:])
        n = h
    return a


def _router_body(x_ref, w_ref, bias_ref, wT_out_ref, idxT_out_ref,
                 counts_ref, stats_ref):
    i = pl.program_id(0)
    nsteps = pl.num_programs(0)

    w = w_ref[...]
    x = x_ref[...]
    lt = jax.lax.dot_general(w, x, (((1,), (1,)), ((), ())),
                             preferred_element_type=jnp.float32)
    lt = lt + bias_ref[...]

    sub_f = jax.lax.broadcasted_iota(jnp.int32, (N_EXPERTS, BLOCK), 0).astype(
        jnp.float32)
    cur = lt
    m_rows = []
    idx_rows = []
    for k in range(TOP_K):
        m = _tree(cur, jnp.maximum)
        t = cur == m
        idx = _tree(jnp.where(t, sub_f, jnp.float32(N_EXPERTS)),
                    jnp.minimum)
        m_rows.append(m)
        idx_rows.append(idx)
        cur = jnp.where(t & (sub_f == idx), -jnp.inf, cur)

    vals = jnp.concatenate(m_rows, axis=0)
    e = jnp.exp(vals - vals[:1])
    wT_out_ref[...] = e / _tree(e, jnp.add)
    idxT_out_ref[...] = jnp.concatenate(idx_rows, axis=0).astype(jnp.int32)

    block_counts = jnp.sum(jnp.where(cur == -jnp.inf, 1.0, 0.0),
                           axis=1, keepdims=True)

    @pl.when(i == 0)
    def _init():
        counts_ref[...] = block_counts

    @pl.when(i != 0)
    def _acc():
        counts_ref[...] = counts_ref[...] + block_counts

    @pl.when(i == nsteps - 1)
    def _stats():
        c = counts_ref[...]
        mean = _tree(c, jnp.add) / N_EXPERTS
        d = c - mean
        var = _tree(d * d, jnp.add) / (N_EXPERTS - 1)
        lb = jnp.sqrt(var) / (mean + 1e-6)
        cmax = _tree(c, jnp.maximum)
        cmin = _tree(c, jnp.minimum)
        l0 = jax.lax.broadcasted_iota(jnp.int32, (1, 128), 1)
        stats_ref[...] = (jnp.where(l0 == 0, lb, 0.0)
                          + jnp.where(l0 == 1, cmax, 0.0)
                          + jnp.where(l0 == 2, cmin, 0.0))


def kernel(x, W, expert_bias):
    b, s, d = x.shape
    nt = b * s
    x_flat = x.reshape(nt, d)
    bias = expert_bias.reshape(N_EXPERTS, 1)

    grid = (nt // BLOCK,)
    wT_out, idxT_out, counts, stats = pl.pallas_call(
        _router_body,
        grid=grid,
        in_specs=[
            pl.BlockSpec((BLOCK, d), lambda i: (i, 0)),
            pl.BlockSpec((N_EXPERTS, d), lambda i: (0, 0)),
            pl.BlockSpec((N_EXPERTS, 1), lambda i: (0, 0)),
        ],
        out_specs=[
            pl.BlockSpec((TOP_K, BLOCK), lambda i: (0, i)),
            pl.BlockSpec((TOP_K, BLOCK), lambda i: (0, i)),
            pl.BlockSpec((N_EXPERTS, 1), lambda i: (0, 0)),
            pl.BlockSpec((1, 128), lambda i: (0, 0)),
        ],
        out_shape=[
            jax.ShapeDtypeStruct((TOP_K, nt), jnp.float32),
            jax.ShapeDtypeStruct((TOP_K, nt), jnp.int32),
            jax.ShapeDtypeStruct((N_EXPERTS, 1), jnp.float32),
            jax.ShapeDtypeStruct((1, 128), jnp.float32),
        ],
        compiler_params=pltpu.CompilerParams(
            dimension_semantics=("arbitrary",),
        ),
    )(x_flat, W, bias)

    routing_weights = wT_out.T.reshape(b, s, TOP_K)
    expert_indices = idxT_out.T.reshape(b, s, TOP_K)
    expert_counts = counts[:, 0]
    load_balance = stats[0, 0]
    cmax = stats[0, 1]
    cmin = stats[0, 2]
    expected_load = jnp.asarray(nt * TOP_K / N_EXPERTS, dtype=jnp.float32)
    return (routing_weights, expert_indices, expert_counts, load_balance,
            cmax, cmin, expected_load)
